# baseline (speedup 1.0000x reference)
import functools

import jax
import jax.numpy as jnp
from jax import lax
from jax.experimental import pallas as pl
from jax.experimental.pallas import tpu as pltpu
from jax.experimental.pallas import tpu_sc as plsc

_NC = 2
_NS = 16
_K = 96
_JUNK = 16


def _sc_aggregate(x, srcp, dstp, chunks):
    n, d = x.shape
    dh = d // _NC
    rows_per_tile = (n // _NS) & ~7
    rows_tail = n - _NS * rows_per_tile
    nsteps = chunks // 2

    mesh = plsc.VectorSubcoreMesh(core_axis_name="c", subcore_axis_name="s")

    @functools.partial(
        pl.kernel,
        out_type=jax.ShapeDtypeStruct((n, d), jnp.float32),
        mesh=mesh,
        scratch_types=[
            pltpu.VMEM_SHARED((n + _JUNK, dh), jnp.float32),
            pltpu.VMEM((chunks * _K,), jnp.int32),
            pltpu.VMEM((chunks * _K,), jnp.int32),
            pltpu.VMEM((_K, dh), jnp.float32),
            pltpu.VMEM((_K, dh), jnp.float32),
            pltpu.SemaphoreType.DMA,
            pltpu.SemaphoreType.DMA,
            pltpu.SemaphoreType.DMA,
            pltpu.SemaphoreType.DMA,
            pltpu.SemaphoreType.DMA,
            pltpu.SemaphoreType.DMA,
        ],
    )
    def agg_kernel(x_hbm, src_hbm, dst_hbm, out_hbm, acc, sidx, didx, g0,
                   g1, sis, dis, gs0, gs1, ss0, ss1):
        c = lax.axis_index("c")
        s = lax.axis_index("s")
        col0 = c * dh
        row0 = s * rows_per_tile
        pltpu.async_copy(src_hbm.at[s], sidx, sis)
        pltpu.async_copy(dst_hbm.at[s], didx, dis)
        pltpu.sync_copy(x_hbm.at[pl.ds(row0, rows_per_tile),
                                 pl.ds(col0, dh)],
                        acc.at[pl.ds(row0, rows_per_tile)])
        if rows_tail:
            @pl.when(s == _NS - 1)
            def _():
                t0 = _NS * rows_per_tile
                pltpu.sync_copy(x_hbm.at[pl.ds(t0, rows_tail),
                                         pl.ds(col0, dh)],
                                acc.at[pl.ds(t0, rows_tail)])
        pltpu.make_async_copy(src_hbm.at[s], sidx, sis).wait()
        pltpu.make_async_copy(dst_hbm.at[s], didx, dis).wait()
        plsc.subcore_barrier()

        pltpu.async_copy(x_hbm.at[sidx.at[pl.ds(0, _K)], pl.ds(col0, dh)], g0, gs0)
        pltpu.async_copy(x_hbm.at[sidx.at[pl.ds(_K, _K)], pl.ds(col0, dh)], g1, gs1)

        def body(j, carry):
            j0 = 2 * j
            pltpu.make_async_copy(x_hbm.at[sidx.at[pl.ds(j0 * _K, _K)], pl.ds(col0, dh)],
                                  g0, gs0).wait()
            pltpu.async_copy(g0, acc.at[didx.at[pl.ds(j0 * _K, _K)]], ss0,
                             add=True)

            pltpu.make_async_copy(x_hbm.at[sidx.at[pl.ds((j0 + 1) * _K, _K)],
                                           pl.ds(col0, dh)], g1,
                                  gs1).wait()
            pltpu.async_copy(g1, acc.at[didx.at[pl.ds((j0 + 1) * _K, _K)]],
                             ss1, add=True)

            pltpu.make_async_copy(g0, acc.at[didx.at[pl.ds(j0 * _K, _K)]],
                                  ss0).wait()

            @pl.when(j0 + 2 < chunks)
            def _():
                pltpu.async_copy(x_hbm.at[sidx.at[pl.ds((j0 + 2) * _K, _K)],
                                          pl.ds(col0, dh)], g0, gs0)

            pltpu.make_async_copy(g1, acc.at[didx.at[pl.ds((j0 + 1) * _K, _K)]],
                                  ss1).wait()

            @pl.when(j0 + 3 < chunks)
            def _():
                pltpu.async_copy(x_hbm.at[sidx.at[pl.ds((j0 + 3) * _K, _K)],
                                          pl.ds(col0, dh)], g1, gs1)

            return carry

        lax.fori_loop(0, nsteps, body, 0)
        plsc.subcore_barrier()
        pltpu.sync_copy(acc.at[pl.ds(row0, rows_per_tile)],
                        out_hbm.at[pl.ds(row0, rows_per_tile),
                                   pl.ds(col0, dh)])
        if rows_tail:
            @pl.when(s == _NS - 1)
            def _():
                t0 = _NS * rows_per_tile
                pltpu.sync_copy(acc.at[pl.ds(t0, rows_tail)],
                                out_hbm.at[pl.ds(t0, rows_tail),
                                           pl.ds(col0, dh)])

    return agg_kernel(x, srcp, dstp)


def _tc_fold(w1, b1, w2, b2):
    d, h = w1.shape
    d2 = w2.shape[1]

    def fold_kernel(w1_ref, b1_ref, w2_ref, b2_ref, wc_ref, bc_ref):
        wc_ref[...] = jnp.dot(w1_ref[...], w2_ref[...],
                              preferred_element_type=jnp.float32)
        bc_ref[...] = jnp.dot(b1_ref[...], w2_ref[...],
                              preferred_element_type=jnp.float32) + b2_ref[...]

    return pl.pallas_call(
        fold_kernel,
        out_shape=(jax.ShapeDtypeStruct((d, d2), jnp.float32),
                   jax.ShapeDtypeStruct((1, d2), jnp.float32)),
    )(w1, b1.reshape(1, h), w2, b2.reshape(1, d2))


def _tc_dense(acc, x, eps, wc, bc, bn):
    n, d = acc.shape
    grid = (n // bn,)

    def dense_kernel(eps_ref, a_ref, x_ref, wc_ref, bc_ref, out_ref):
        h = a_ref[...] + eps_ref[0] * x_ref[...]
        r = jnp.dot(h, wc_ref[...], preferred_element_type=jnp.float32)
        out_ref[...] = jnp.maximum(r + bc_ref[...], 0.0)

    return pl.pallas_call(
        dense_kernel,
        grid=grid,
        in_specs=[
            pl.BlockSpec(memory_space=pltpu.SMEM),
            pl.BlockSpec((bn, d), lambda i: (i, 0)),
            pl.BlockSpec((bn, d), lambda i: (i, 0)),
            pl.BlockSpec((d, d), lambda i: (0, 0)),
            pl.BlockSpec((1, d), lambda i: (0, 0)),
        ],
        out_specs=pl.BlockSpec((bn, d), lambda i: (i, 0)),
        out_shape=jax.ShapeDtypeStruct((n, d), jnp.float32),
    )(eps.reshape(1), acc, x, wc, bc)


def kernel(x, edge_index, eps1, W11, b11, W12, b12, eps2, W21, b21, W22,
           b22):
    n, d = x.shape
    dh = d // 2
    e = edge_index.shape[1]
    src = edge_index[0]
    dst = edge_index[1]

    e_per_tile = e // _NS
    chunks = -(-e_per_tile // _K)
    chunks += chunks % 2
    pad = chunks * _K - e_per_tile
    src2 = src.reshape(_NS, e_per_tile)
    dst2 = dst.reshape(_NS, e_per_tile)
    if pad:
        pad_src = jnp.broadcast_to((jnp.arange(pad, dtype=jnp.int32) * 64)
                                   % n, (_NS, pad))
        pad_dst = jnp.broadcast_to(
            n + jnp.arange(pad, dtype=jnp.int32) % _JUNK, (_NS, pad))
        src2 = jnp.concatenate([src2, pad_src], axis=1)
        dst2 = jnp.concatenate([dst2, pad_dst], axis=1)
    srcp = src2.reshape(_NS, chunks * _K)
    dstp = dst2.reshape(_NS, chunks * _K)

    wc1, bc1 = _tc_fold(W11, b11, W12, b12)
    wc2, bc2 = _tc_fold(W21, b21, W22, b22)

    bn = 1000
    acc1 = _sc_aggregate(x, srcp, dstp, chunks)
    x1 = _tc_dense(acc1, x, eps1, wc1, bc1, bn)
    acc2 = _sc_aggregate(x1, srcp, dstp, chunks)
    x2 = _tc_dense(acc2, x1, eps2, wc2, bc2, bn)

    return jnp.concatenate([x1[:, :, None], x2[:, :, None]], axis=2)


# speedup vs baseline: 1.1088x; 1.1088x over previous
import functools

import jax
import jax.numpy as jnp
from jax import lax
from jax.experimental import pallas as pl
from jax.experimental.pallas import tpu as pltpu
from jax.experimental.pallas import tpu_sc as plsc

_NC = 2
_NS = 16
_K = 96
_JUNK = 16


def _sc_aggregate(x, srcp, dstp, chunks):
    n, d = x.shape
    dh = d // _NC
    rows_per_tile = (n // _NS) & ~7
    rows_tail = n - _NS * rows_per_tile
    nsteps = chunks // 2

    mesh = plsc.VectorSubcoreMesh(core_axis_name="c", subcore_axis_name="s")

    @functools.partial(
        pl.kernel,
        out_type=jax.ShapeDtypeStruct((n, d), jnp.float32),
        mesh=mesh,
        scratch_types=[
            pltpu.VMEM_SHARED((n + _JUNK, dh), jnp.float32),
            pltpu.VMEM((chunks * _K,), jnp.int32),
            pltpu.VMEM((chunks * _K,), jnp.int32),
            pltpu.VMEM((_K, dh), jnp.float32),
            pltpu.VMEM((_K, dh), jnp.float32),
            pltpu.SemaphoreType.DMA,
            pltpu.SemaphoreType.DMA,
            pltpu.SemaphoreType.DMA,
            pltpu.SemaphoreType.DMA,
        ],
    )
    def agg_kernel(x_hbm, src_hbm, dst_hbm, out_hbm, acc, sidx, didx, g0,
                   g1, sis, dis, gs0, gs1):
        c = lax.axis_index("c")
        s = lax.axis_index("s")
        col0 = c * dh
        row0 = s * rows_per_tile
        pltpu.async_copy(src_hbm.at[s], sidx, sis)
        pltpu.async_copy(dst_hbm.at[s], didx, dis)
        pltpu.sync_copy(x_hbm.at[pl.ds(row0, rows_per_tile),
                                 pl.ds(col0, dh)],
                        acc.at[pl.ds(row0, rows_per_tile)])
        if rows_tail:
            @pl.when(s == _NS - 1)
            def _():
                t0 = _NS * rows_per_tile
                pltpu.sync_copy(x_hbm.at[pl.ds(t0, rows_tail),
                                         pl.ds(col0, dh)],
                                acc.at[pl.ds(t0, rows_tail)])
        pltpu.make_async_copy(src_hbm.at[s], sidx, sis).wait()
        pltpu.make_async_copy(dst_hbm.at[s], didx, dis).wait()
        plsc.subcore_barrier()

        pltpu.async_copy(x_hbm.at[sidx.at[pl.ds(0, _K)], pl.ds(col0, dh)], g0, gs0)
        pltpu.async_copy(x_hbm.at[sidx.at[pl.ds(_K, _K)], pl.ds(col0, dh)], g1, gs1)

        def body(j, carry):
            j0 = 2 * j
            pltpu.make_async_copy(x_hbm.at[sidx.at[pl.ds(j0 * _K, _K)], pl.ds(col0, dh)],
                                  g0, gs0).wait()
            pltpu.sync_copy(g0, acc.at[didx.at[pl.ds(j0 * _K, _K)]], add=True)

            @pl.when(j0 + 2 < chunks)
            def _():
                pltpu.async_copy(x_hbm.at[sidx.at[pl.ds((j0 + 2) * _K, _K)],
                                          pl.ds(col0, dh)], g0, gs0)

            pltpu.make_async_copy(x_hbm.at[sidx.at[pl.ds((j0 + 1) * _K, _K)],
                                           pl.ds(col0, dh)], g1,
                                  gs1).wait()
            pltpu.sync_copy(g1, acc.at[didx.at[pl.ds((j0 + 1) * _K, _K)]], add=True)

            @pl.when(j0 + 3 < chunks)
            def _():
                pltpu.async_copy(x_hbm.at[sidx.at[pl.ds((j0 + 3) * _K, _K)],
                                          pl.ds(col0, dh)], g1, gs1)

            return carry

        lax.fori_loop(0, nsteps, body, 0)
        plsc.subcore_barrier()
        pltpu.sync_copy(acc.at[pl.ds(row0, rows_per_tile)],
                        out_hbm.at[pl.ds(row0, rows_per_tile),
                                   pl.ds(col0, dh)])
        if rows_tail:
            @pl.when(s == _NS - 1)
            def _():
                t0 = _NS * rows_per_tile
                pltpu.sync_copy(acc.at[pl.ds(t0, rows_tail)],
                                out_hbm.at[pl.ds(t0, rows_tail),
                                           pl.ds(col0, dh)])

    return agg_kernel(x, srcp, dstp)


def _tc_fold(w1, b1, w2, b2):
    d, h = w1.shape
    d2 = w2.shape[1]

    def fold_kernel(w1_ref, b1_ref, w2_ref, b2_ref, wc_ref, bc_ref):
        wc_ref[...] = jnp.dot(w1_ref[...], w2_ref[...],
                              preferred_element_type=jnp.float32)
        bc_ref[...] = jnp.dot(b1_ref[...], w2_ref[...],
                              preferred_element_type=jnp.float32) + b2_ref[...]

    return pl.pallas_call(
        fold_kernel,
        out_shape=(jax.ShapeDtypeStruct((d, d2), jnp.float32),
                   jax.ShapeDtypeStruct((1, d2), jnp.float32)),
    )(w1, b1.reshape(1, h), w2, b2.reshape(1, d2))


def _tc_dense(acc, x, eps, wc, bc, bn):
    n, d = acc.shape
    grid = (n // bn,)

    def dense_kernel(eps_ref, a_ref, x_ref, wc_ref, bc_ref, out_ref):
        h = a_ref[...] + eps_ref[0] * x_ref[...]
        r = jnp.dot(h, wc_ref[...], preferred_element_type=jnp.float32)
        out_ref[...] = jnp.maximum(r + bc_ref[...], 0.0)

    return pl.pallas_call(
        dense_kernel,
        grid=grid,
        in_specs=[
            pl.BlockSpec(memory_space=pltpu.SMEM),
            pl.BlockSpec((bn, d), lambda i: (i, 0)),
            pl.BlockSpec((bn, d), lambda i: (i, 0)),
            pl.BlockSpec((d, d), lambda i: (0, 0)),
            pl.BlockSpec((1, d), lambda i: (0, 0)),
        ],
        out_specs=pl.BlockSpec((bn, d), lambda i: (i, 0)),
        out_shape=jax.ShapeDtypeStruct((n, d), jnp.float32),
    )(eps.reshape(1), acc, x, wc, bc)


def kernel(x, edge_index, eps1, W11, b11, W12, b12, eps2, W21, b21, W22,
           b22):
    n, d = x.shape
    dh = d // 2
    e = edge_index.shape[1]
    src = edge_index[0]
    dst = edge_index[1]

    e_per_tile = e // _NS
    chunks = -(-e_per_tile // _K)
    chunks += chunks % 2
    pad = chunks * _K - e_per_tile
    src2 = src.reshape(_NS, e_per_tile)
    dst2 = dst.reshape(_NS, e_per_tile)
    if pad:
        pad_src = jnp.broadcast_to((jnp.arange(pad, dtype=jnp.int32) * 64)
                                   % n, (_NS, pad))
        pad_dst = jnp.broadcast_to(
            n + jnp.arange(pad, dtype=jnp.int32) % _JUNK, (_NS, pad))
        src2 = jnp.concatenate([src2, pad_src], axis=1)
        dst2 = jnp.concatenate([dst2, pad_dst], axis=1)
    srcp = src2.reshape(_NS, chunks * _K)
    dstp = dst2.reshape(_NS, chunks * _K)

    wc1, bc1 = _tc_fold(W11, b11, W12, b12)
    wc2, bc2 = _tc_fold(W21, b21, W22, b22)

    bn = 1000
    acc1 = _sc_aggregate(x, srcp, dstp, chunks)
    x1 = _tc_dense(acc1, x, eps1, wc1, bc1, bn)
    acc2 = _sc_aggregate(x1, srcp, dstp, chunks)
    x2 = _tc_dense(acc2, x1, eps2, wc2, bc2, bn)

    return jnp.concatenate([x1[:, :, None], x2[:, :, None]], axis=2)
